# Initial kernel scaffold; baseline (speedup 1.0000x reference)
import jax
import jax.numpy as jnp
from jax.experimental import pallas as pl


def kernel(query, query_pos, reference_points, spatial_shapes, level_start_index, W_value, b_value, W_so, b_so, W_aw, b_aw, W_out, b_out):
    raise NotImplementedError("write your pallas kernel here")


# speedup vs baseline: 2069.5745x; 2069.5745x over previous
import functools

import jax
import jax.numpy as jnp
import numpy as np
from jax import lax
from jax.experimental import pallas as pl
from jax.experimental.pallas import tpu as pltpu
from jax.experimental.pallas import tpu_sc as plsc

D = 256
H = 8
HD = 32
NBQ = 2
NP = 4
GH, GW = 100, 100
NQ = GH * GW
RPQ = NBQ * NP * 4

QB = 400
NQB = NQ // QB

NW = 32
CHUNK_Q = 20
ROWS = CHUNK_Q * RPQ
NIDX = ROWS // 128
QPT = NQ // (NW // H)
NCHUNK = QPT // CHUNK_Q

_PERM_SO = np.empty(128, np.int64)
for _h in range(H):
    for _b in range(NBQ):
        for _p in range(NP):
            _new = _h * 8 + _b * 4 + _p
            _PERM_SO[_new] = _h * 16 + _b * 8 + _p * 2 + 0
            _PERM_SO[64 + _new] = _h * 16 + _b * 8 + _p * 2 + 1
_PERM_AW = np.empty(64, np.int64)
_PMAT = np.zeros((64, 64), np.float32)
for _h in range(H):
    for _b in range(NBQ):
        for _p in range(NP):
            _PERM_AW[_p * 16 + _h * 2 + _b] = _h * 8 + _b * 4 + _p
            _PMAT[_p * 16 + _h * 2 + _b, _h * 8 + _b * 4 + _p] = 1.0
_BSEL = (np.arange(64) // 4) % 2


def _tc1_body(q_ref, qp_ref, xb_ref, yb_ref, Wv_ref, bv_ref, Wso_ref,
              bso_ref, Waw_ref, baw_ref, P_ref, vh_ref, idx_ref, w_ref):
    q = q_ref[...]
    v = jnp.dot(q, Wv_ref[...], preferred_element_type=jnp.float32) + bv_ref[...]
    qc = jnp.concatenate([q, q + qp_ref[...]], axis=-1)
    so = jnp.dot(qc, Wso_ref[...], preferred_element_type=jnp.float32) + bso_ref[...]
    awl = jnp.dot(qc, Waw_ref[...], preferred_element_type=jnp.float32) + baw_ref[...]
    m = jnp.maximum(jnp.maximum(awl[:, 0:16], awl[:, 16:32]),
                    jnp.maximum(awl[:, 32:48], awl[:, 48:64]))
    e = jnp.exp(awl - jnp.concatenate([m, m, m, m], axis=-1))
    s = e[:, 0:16] + e[:, 16:32] + e[:, 32:48] + e[:, 48:64]
    aw = e / jnp.concatenate([s, s, s, s], axis=-1)
    aw = jnp.dot(aw, P_ref[...], preferred_element_type=jnp.float32)

    xf = so[:, 0:64] + xb_ref[...]
    yf = so[:, 64:128] + yb_ref[...]
    x0 = jnp.floor(xf)
    fx = xf - x0
    cb = jnp.clip(x0, 0.0, 98.0)
    wc0 = jnp.where(x0 == cb, 1.0 - fx, jnp.where(x0 + 1.0 == cb, fx, 0.0))
    wc1 = jnp.where(x0 == cb, fx, jnp.where(x0 == cb + 1.0, 1.0 - fx, 0.0))
    y0 = jnp.floor(yf)
    fy = yf - y0
    rb0 = jnp.clip(y0, 0.0, 99.0)
    rb1 = jnp.clip(y0 + 1.0, 0.0, 99.0)
    wr0 = jnp.where(y0 == rb0, 1.0 - fy, 0.0)
    wr1 = jnp.where(y0 + 1.0 == rb1, fy, 0.0)
    idxA = rb0 * 100.0 + cb
    idxB = rb1 * 100.0 + cb
    half = 0.5 * aw
    wA0 = half * wr0 * wc0
    wA1 = half * wr0 * wc1
    wB0 = half * wr1 * wc0
    wB1 = half * wr1 * wc1
    for h in range(H):
        lo, hi = h * 8, h * 8 + 8
        ia = idxA[:, lo:hi]
        ib = idxB[:, lo:hi]
        idx_ref[h] = jnp.concatenate([ia, ia + 1.0, ib, ib + 1.0],
                                     axis=-1).astype(jnp.int32)
        w_ref[h] = jnp.concatenate([wA0[:, lo:hi], wA1[:, lo:hi],
                                    wB0[:, lo:hi], wB1[:, lo:hi]], axis=-1)
        vh_ref[h] = v[:, h * HD:(h + 1) * HD]


def _tc1_call(q, qp, xb, yb, Wv, bv, Wso, bso, Waw, baw, P, interpret=False):
    return pl.pallas_call(
        _tc1_body,
        grid=(NQB,),
        in_specs=[
            pl.BlockSpec((QB, D), lambda i: (i, 0)),
            pl.BlockSpec((QB, D), lambda i: (i, 0)),
            pl.BlockSpec((QB, 64), lambda i: (i, 0)),
            pl.BlockSpec((QB, 64), lambda i: (i, 0)),
            pl.BlockSpec((D, D), lambda i: (0, 0)),
            pl.BlockSpec((1, D), lambda i: (0, 0)),
            pl.BlockSpec((2 * D, 128), lambda i: (0, 0)),
            pl.BlockSpec((1, 128), lambda i: (0, 0)),
            pl.BlockSpec((2 * D, 64), lambda i: (0, 0)),
            pl.BlockSpec((1, 64), lambda i: (0, 0)),
            pl.BlockSpec((64, 64), lambda i: (0, 0)),
        ],
        out_specs=[
            pl.BlockSpec((H, QB, HD), lambda i: (0, i, 0)),
            pl.BlockSpec((H, QB, RPQ), lambda i: (0, i, 0)),
            pl.BlockSpec((H, QB, RPQ), lambda i: (0, i, 0)),
        ],
        out_shape=[
            jax.ShapeDtypeStruct((H, NQ, HD), jnp.float32),
            jax.ShapeDtypeStruct((H, NQ, RPQ), jnp.int32),
            jax.ShapeDtypeStruct((H, NQ, RPQ), jnp.float32),
        ],
        interpret=interpret,
    )(q, qp, xb, yb, Wv, bv, Wso, bso, Waw, baw, P)


def _sc_body(vh_hbm, idx_hbm, w_hbm, out_hbm, idx_v, w_v, rows_v, out_v, sem):
    wid = lax.axis_index("s") * 2 + lax.axis_index("c")
    head = wid % H
    q0 = (wid // H) * QPT

    def chunk_body(ci, carry):
        qs = q0 + ci * CHUNK_Q
        off = head * (NQ * RPQ) + qs * RPQ
        pltpu.sync_copy(idx_hbm.at[pl.ds(off, ROWS)], idx_v)
        pltpu.sync_copy(w_hbm.at[pl.ds(off, ROWS)], w_v)
        descs = []
        for j in range(NIDX):
            descs.append(pltpu.async_copy(
                vh_hbm.at[head].at[idx_v.at[pl.ds(j * 128, 128)]],
                rows_v.at[pl.ds(j * 128, 128)], sem))
        for d in descs:
            d.wait()

        def q_body(qi, c2):
            base = qi * RPQ
            wv0 = w_v[pl.ds(base, 16)]
            wv1 = w_v[pl.ds(base + 16, 16)]
            acc0 = jnp.zeros((16,), jnp.float32)
            acc1 = jnp.zeros((16,), jnp.float32)
            for k in range(RPQ):
                src = wv0 if k < 16 else wv1
                wsp = lax.gather(
                    src, jnp.full((16, 1), k % 16, jnp.int32),
                    dimension_numbers=lax.GatherDimensionNumbers(
                        offset_dims=(), collapsed_slice_dims=(0,),
                        start_index_map=(0,)),
                    slice_sizes=(1,),
                    mode=lax.GatherScatterMode.PROMISE_IN_BOUNDS)
                acc0 = acc0 + wsp * rows_v[base + k, pl.ds(0, 16)]
                acc1 = acc1 + wsp * rows_v[base + k, pl.ds(16, 16)]
            out_v[pl.ds(base, 16)] = acc0
            out_v[pl.ds(base + 16, 16)] = acc1
            return c2

        lax.fori_loop(0, CHUNK_Q, q_body, 0)
        pltpu.sync_copy(out_v, out_hbm.at[pl.ds(head * (NQ * HD) + qs * HD,
                                                CHUNK_Q * HD)])
        return carry

    lax.fori_loop(0, NCHUNK, chunk_body, 0)


@functools.lru_cache(maxsize=1)
def _sc_call():
    return pl.kernel(
        _sc_body,
        out_type=jax.ShapeDtypeStruct((H * NQ * HD,), jnp.float32),
        mesh=plsc.VectorSubcoreMesh(core_axis_name="c", subcore_axis_name="s"),
        scratch_types=[
            pltpu.VMEM((ROWS,), jnp.int32),
            pltpu.VMEM((ROWS,), jnp.float32),
            pltpu.VMEM((ROWS, HD), jnp.float32),
            pltpu.VMEM((CHUNK_Q * HD,), jnp.float32),
            pltpu.SemaphoreType.DMA,
        ],
        compiler_params=pltpu.CompilerParams(use_tc_tiling_on_sc=False),
    )


def _tc2_body(q_ref, s_ref, Wo_ref, bo_ref, o_ref):
    acc = q_ref[...] + bo_ref[...]
    for h in range(H):
        acc = acc + jnp.dot(s_ref[h], Wo_ref[h],
                            preferred_element_type=jnp.float32)
    o_ref[...] = acc


def _tc2_call(q, sout, Wo, bo, interpret=False):
    return pl.pallas_call(
        _tc2_body,
        grid=(NQB,),
        in_specs=[
            pl.BlockSpec((QB, D), lambda i: (i, 0)),
            pl.BlockSpec((H, QB, HD), lambda i: (0, i, 0)),
            pl.BlockSpec((H, HD, D), lambda i: (0, 0, 0)),
            pl.BlockSpec((1, D), lambda i: (0, 0)),
        ],
        out_specs=pl.BlockSpec((QB, D), lambda i: (i, 0)),
        out_shape=jax.ShapeDtypeStruct((NQ, D), jnp.float32),
        interpret=interpret,
    )(q, sout, Wo, bo)


def kernel(query, query_pos, reference_points, spatial_shapes,
           level_start_index, W_value, b_value, W_so, b_so, W_aw, b_aw,
           W_out, b_out):
    q2 = query[0]
    qp2 = query_pos[0]
    refx = reference_points[:, :, 0, 0].T
    refy = reference_points[:, :, 0, 1].T
    xb = refx[:, _BSEL] * 100.0 - 0.5
    yb = refy[:, _BSEL] * 100.0 - 0.5
    Wso_p = W_so[:, _PERM_SO]
    bso_p = b_so[_PERM_SO][None, :]
    Waw_p = W_aw[:, _PERM_AW]
    baw_p = b_aw[_PERM_AW][None, :]
    P = jnp.asarray(_PMAT)
    vheads, idx, w = _tc1_call(q2, qp2, xb, yb, W_value, b_value[None, :],
                               Wso_p, bso_p, Waw_p, baw_p, P)
    sout = _sc_call()(vheads, idx.reshape(-1), w.reshape(-1))
    sout = sout.reshape(H, NQ, HD)
    out = _tc2_call(q2, sout, W_out.reshape(H, HD, D), b_out[None, :])
    return out[None]


# speedup vs baseline: 3272.7084x; 1.5813x over previous
import functools

import jax
import jax.numpy as jnp
import numpy as np
from jax import lax
from jax.experimental import pallas as pl
from jax.experimental.pallas import tpu as pltpu
from jax.experimental.pallas import tpu_sc as plsc

D = 256
H = 8
HD = 32
NBQ = 2
NP = 4
GH, GW = 100, 100
NQ = GH * GW
RPQ = NBQ * NP * 4

QB = 400
NQB = NQ // QB

NW = 32
CHUNK_Q = 20
ROWS = CHUNK_Q * RPQ
NIDX = ROWS // 128
QPT = NQ // (NW // H)
NCHUNK = QPT // CHUNK_Q

_PERM_SO = np.empty(128, np.int64)
for _h in range(H):
    for _b in range(NBQ):
        for _p in range(NP):
            _new = _h * 8 + _b * 4 + _p
            _PERM_SO[_new] = _h * 16 + _b * 8 + _p * 2 + 0
            _PERM_SO[64 + _new] = _h * 16 + _b * 8 + _p * 2 + 1
_PERM_AW = np.empty(64, np.int64)
_PMAT = np.zeros((64, 64), np.float32)
for _h in range(H):
    for _b in range(NBQ):
        for _p in range(NP):
            _PERM_AW[_p * 16 + _h * 2 + _b] = _h * 8 + _b * 4 + _p
            _PMAT[_p * 16 + _h * 2 + _b, _h * 8 + _b * 4 + _p] = 1.0
_BSEL = (np.arange(64) // 4) % 2


def _tc1_body(q_ref, qp_ref, xb_ref, yb_ref, Wv_ref, bv_ref, Wso_ref,
              bso_ref, Waw_ref, baw_ref, P_ref, vh_ref, idx_ref, w_ref):
    q = q_ref[...]
    v = jnp.dot(q, Wv_ref[...], preferred_element_type=jnp.float32) + bv_ref[...]
    qc = jnp.concatenate([q, q + qp_ref[...]], axis=-1)
    so = jnp.dot(qc, Wso_ref[...], preferred_element_type=jnp.float32) + bso_ref[...]
    awl = jnp.dot(qc, Waw_ref[...], preferred_element_type=jnp.float32) + baw_ref[...]
    m = jnp.maximum(jnp.maximum(awl[:, 0:16], awl[:, 16:32]),
                    jnp.maximum(awl[:, 32:48], awl[:, 48:64]))
    e = jnp.exp(awl - jnp.concatenate([m, m, m, m], axis=-1))
    s = e[:, 0:16] + e[:, 16:32] + e[:, 32:48] + e[:, 48:64]
    aw = e / jnp.concatenate([s, s, s, s], axis=-1)
    aw = jnp.dot(aw, P_ref[...], preferred_element_type=jnp.float32)

    xf = so[:, 0:64] + xb_ref[...]
    yf = so[:, 64:128] + yb_ref[...]
    x0 = jnp.floor(xf)
    fx = xf - x0
    cb = jnp.clip(x0, 0.0, 98.0)
    wc0 = jnp.where(x0 == cb, 1.0 - fx, jnp.where(x0 + 1.0 == cb, fx, 0.0))
    wc1 = jnp.where(x0 == cb, fx, jnp.where(x0 == cb + 1.0, 1.0 - fx, 0.0))
    y0 = jnp.floor(yf)
    fy = yf - y0
    rb0 = jnp.clip(y0, 0.0, 99.0)
    rb1 = jnp.clip(y0 + 1.0, 0.0, 99.0)
    wr0 = jnp.where(y0 == rb0, 1.0 - fy, 0.0)
    wr1 = jnp.where(y0 + 1.0 == rb1, fy, 0.0)
    idxA = rb0 * 100.0 + cb
    idxB = rb1 * 100.0 + cb
    half = 0.5 * aw
    wA0 = half * wr0 * wc0
    wA1 = half * wr0 * wc1
    wB0 = half * wr1 * wc0
    wB1 = half * wr1 * wc1
    for h in range(H):
        lo, hi = h * 8, h * 8 + 8
        ia = idxA[:, lo:hi]
        ib = idxB[:, lo:hi]
        idx_ref[h] = jnp.concatenate([ia, ia + 1.0, ib, ib + 1.0],
                                     axis=-1).astype(jnp.int32)
        w_ref[h] = jnp.concatenate([wA0[:, lo:hi], wA1[:, lo:hi],
                                    wB0[:, lo:hi], wB1[:, lo:hi]], axis=-1)
        vh_ref[h] = v[:, h * HD:(h + 1) * HD]


def _tc1_call(q, qp, xb, yb, Wv, bv, Wso, bso, Waw, baw, P, interpret=False):
    return pl.pallas_call(
        _tc1_body,
        grid=(NQB,),
        in_specs=[
            pl.BlockSpec((QB, D), lambda i: (i, 0)),
            pl.BlockSpec((QB, D), lambda i: (i, 0)),
            pl.BlockSpec((QB, 64), lambda i: (i, 0)),
            pl.BlockSpec((QB, 64), lambda i: (i, 0)),
            pl.BlockSpec((D, D), lambda i: (0, 0)),
            pl.BlockSpec((1, D), lambda i: (0, 0)),
            pl.BlockSpec((2 * D, 128), lambda i: (0, 0)),
            pl.BlockSpec((1, 128), lambda i: (0, 0)),
            pl.BlockSpec((2 * D, 64), lambda i: (0, 0)),
            pl.BlockSpec((1, 64), lambda i: (0, 0)),
            pl.BlockSpec((64, 64), lambda i: (0, 0)),
        ],
        out_specs=[
            pl.BlockSpec((H, QB, HD), lambda i: (0, i, 0)),
            pl.BlockSpec((H, QB, RPQ), lambda i: (0, i, 0)),
            pl.BlockSpec((H, QB, RPQ), lambda i: (0, i, 0)),
        ],
        out_shape=[
            jax.ShapeDtypeStruct((H, NQ, HD), jnp.float32),
            jax.ShapeDtypeStruct((H, NQ, RPQ), jnp.int32),
            jax.ShapeDtypeStruct((H, NQ, RPQ), jnp.float32),
        ],
        interpret=interpret,
    )(q, qp, xb, yb, Wv, bv, Wso, bso, Waw, baw, P)


def _splat(vec, lane):
    return lax.gather(
        vec, jnp.full((16, 1), lane, jnp.int32),
        dimension_numbers=lax.GatherDimensionNumbers(
            offset_dims=(), collapsed_slice_dims=(0,), start_index_map=(0,)),
        slice_sizes=(1,),
        mode=lax.GatherScatterMode.PROMISE_IN_BOUNDS)


def _sc_body(vh_hbm, idx_hbm, w_hbm, out_hbm,
             idx0, idx1, idx2, w0, w1, w2, r0, r1, r2, out_v,
             sio0, sio1, sio2, sg0, sg1, sg2):
    idxs = (idx0, idx1, idx2)
    ws = (w0, w1, w2)
    rows = (r0, r1, r2)
    sio = (sio0, sio1, sio2)
    sg = (sg0, sg1, sg2)
    wid = lax.axis_index("s") * 2 + lax.axis_index("c")
    head = wid % H
    q0 = (wid // H) * QPT

    def start_io(c, b):
        off = head * (NQ * RPQ) + (q0 + c * CHUNK_Q) * RPQ
        pltpu.async_copy(idx_hbm.at[pl.ds(off, ROWS)], idxs[b], sio[b])
        pltpu.async_copy(w_hbm.at[pl.ds(off, ROWS)], ws[b], sio[b])

    def wait_io(b):
        pltpu.make_async_copy(idx_hbm.at[pl.ds(0, ROWS)], idxs[b], sio[b]).wait()
        pltpu.make_async_copy(w_hbm.at[pl.ds(0, ROWS)], ws[b], sio[b]).wait()

    def fire_g(b):
        for j in range(NIDX):
            pltpu.async_copy(
                vh_hbm.at[head].at[idxs[b].at[pl.ds(j * 128, 128)]],
                rows[b].at[pl.ds(j * 128, 128)], sg[b])

    def wait_g(b):
        for j in range(NIDX):
            pltpu.make_async_copy(
                vh_hbm.at[head].at[idxs[b].at[pl.ds(j * 128, 128)]],
                rows[b].at[pl.ds(j * 128, 128)], sg[b]).wait()

    def compute(c, b):
        rr = rows[b]
        wref = ws[b]

        def q_loop(qi, c2):
            base = qi * RPQ
            wv0 = wref[pl.ds(base, 16)]
            wv1 = wref[pl.ds(base + 16, 16)]
            acc0 = jnp.zeros((16,), jnp.float32)
            acc1 = jnp.zeros((16,), jnp.float32)
            for k in range(RPQ):
                wsp = _splat(wv0 if k < 16 else wv1, k % 16)
                acc0 = acc0 + wsp * rr[base + k, pl.ds(0, 16)]
                acc1 = acc1 + wsp * rr[base + k, pl.ds(16, 16)]
            out_v[pl.ds(base, 16)] = acc0
            out_v[pl.ds(base + 16, 16)] = acc1
            return c2

        lax.fori_loop(0, CHUNK_Q, q_loop, 0)
        pltpu.sync_copy(out_v, out_hbm.at[pl.ds(
            head * (NQ * HD) + (q0 + c * CHUNK_Q) * HD, CHUNK_Q * HD)])

    start_io(0, 0)
    wait_io(0)
    fire_g(0)
    start_io(1, 1)

    def body3(i, carry):
        c = 3 * i
        wait_io(1); fire_g(1); start_io(c + 2, 2)
        wait_g(0); compute(c, 0)
        wait_io(2); fire_g(2); start_io(c + 3, 0)
        wait_g(1); compute(c + 1, 1)
        wait_io(0); fire_g(0); start_io(c + 4, 1)
        wait_g(2); compute(c + 2, 2)
        return carry

    lax.fori_loop(0, (NCHUNK - 2) // 3, body3, 0)
    wait_io(1)
    fire_g(1)
    wait_g(0)
    compute(NCHUNK - 2, 0)
    wait_g(1)
    compute(NCHUNK - 1, 1)


@functools.lru_cache(maxsize=1)
def _sc_call():
    return pl.kernel(
        _sc_body,
        out_type=jax.ShapeDtypeStruct((H * NQ * HD,), jnp.float32),
        mesh=plsc.VectorSubcoreMesh(core_axis_name="c", subcore_axis_name="s"),
        scratch_types=(
            [pltpu.VMEM((ROWS,), jnp.int32)] * 3
            + [pltpu.VMEM((ROWS,), jnp.float32)] * 3
            + [pltpu.VMEM((ROWS, HD), jnp.float32)] * 3
            + [pltpu.VMEM((CHUNK_Q * HD,), jnp.float32)]
            + [pltpu.SemaphoreType.DMA] * 6
        ),
        compiler_params=pltpu.CompilerParams(use_tc_tiling_on_sc=False),
    )


def _tc2_body(q_ref, s_ref, Wo_ref, bo_ref, o_ref):
    acc = q_ref[...] + bo_ref[...]
    for h in range(H):
        acc = acc + jnp.dot(s_ref[h], Wo_ref[h],
                            preferred_element_type=jnp.float32)
    o_ref[...] = acc


def _tc2_call(q, sout, Wo, bo, interpret=False):
    return pl.pallas_call(
        _tc2_body,
        grid=(NQB,),
        in_specs=[
            pl.BlockSpec((QB, D), lambda i: (i, 0)),
            pl.BlockSpec((H, QB, HD), lambda i: (0, i, 0)),
            pl.BlockSpec((H, HD, D), lambda i: (0, 0, 0)),
            pl.BlockSpec((1, D), lambda i: (0, 0)),
        ],
        out_specs=pl.BlockSpec((QB, D), lambda i: (i, 0)),
        out_shape=jax.ShapeDtypeStruct((NQ, D), jnp.float32),
        interpret=interpret,
    )(q, sout, Wo, bo)


def kernel(query, query_pos, reference_points, spatial_shapes,
           level_start_index, W_value, b_value, W_so, b_so, W_aw, b_aw,
           W_out, b_out):
    q2 = query[0]
    qp2 = query_pos[0]
    refx = reference_points[:, :, 0, 0].T
    refy = reference_points[:, :, 0, 1].T
    xb = refx[:, _BSEL] * 100.0 - 0.5
    yb = refy[:, _BSEL] * 100.0 - 0.5
    Wso_p = W_so[:, _PERM_SO]
    bso_p = b_so[_PERM_SO][None, :]
    Waw_p = W_aw[:, _PERM_AW]
    baw_p = b_aw[_PERM_AW][None, :]
    P = jnp.asarray(_PMAT)
    vheads, idx, w = _tc1_call(q2, qp2, xb, yb, W_value, b_value[None, :],
                               Wso_p, bso_p, Waw_p, baw_p, P)
    sout = _sc_call()(vheads, idx.reshape(-1), w.reshape(-1))
    sout = sout.reshape(H, NQ, HD)
    out = _tc2_call(q2, sout, W_out.reshape(H, HD, D), b_out[None, :])
    return out[None]


# speedup vs baseline: 4467.7346x; 1.3651x over previous
import functools

import jax
import jax.numpy as jnp
import numpy as np
from jax import lax
from jax.experimental import pallas as pl
from jax.experimental.pallas import tpu as pltpu
from jax.experimental.pallas import tpu_sc as plsc

D = 256
H = 8
HD = 32
NBQ = 2
NP = 4
GH, GW = 100, 100
NQ = GH * GW
RPQ = NBQ * NP * 4

QB = 400
NQB = NQ // QB

NW = 32
CQ = 4
ROWSC = CQ * D
NCHUNK = NQ // CQ
NCHT = 80

_PERM_SO = np.empty(128, np.int64)
_PERM_AW = np.empty(64, np.int64)
for _h in range(H):
    for _b in range(NBQ):
        for _p in range(NP):
            _s = _p * 16 + _h * 2 + _b
            _PERM_SO[_s] = _h * 16 + _b * 8 + _p * 2 + 0
            _PERM_SO[64 + _s] = _h * 16 + _b * 8 + _p * 2 + 1
            _PERM_AW[_s] = _h * 8 + _b * 4 + _p
_BSEL = np.arange(64) % 2

_S01 = np.zeros((64, 256), np.float32)
_S23 = np.zeros((64, 256), np.float32)
_SC0 = np.zeros((64, 256), np.float32)
_SC1 = np.zeros((64, 256), np.float32)
_SC2 = np.zeros((64, 256), np.float32)
_SC3 = np.zeros((64, 256), np.float32)
_BIAS_IDX = np.zeros((1, 256), np.float32)
for _t in range(256):
    _h = _t // 32
    _m = _t % 32
    _bp = _m // 4
    _c = _m % 4
    _b = _bp // 4
    _p = _bp % 4
    _s = _p * 16 + _h * 2 + _b
    if _c in (0, 1):
        _S01[_s, _t] = 1.0
    else:
        _S23[_s, _t] = 1.0
    (_SC0, _SC1, _SC2, _SC3)[_c][_s, _t] = 1.0
    _BIAS_IDX[0, _t] = _h + (8.0 if _c in (1, 3) else 0.0)


def _tc1_body(q_ref, qp_ref, xb_ref, yb_ref, Wv_ref, bv_ref, Wso_ref,
              bso_ref, Waw_ref, baw_ref, S01_ref, S23_ref, SC0_ref, SC1_ref,
              SC2_ref, SC3_ref, bi_ref, v_ref, idx_ref, w_ref):
    q = q_ref[...]
    v_ref[...] = jnp.dot(q, Wv_ref[...],
                         preferred_element_type=jnp.float32) + bv_ref[...]
    qc = jnp.concatenate([q, q + qp_ref[...]], axis=-1)
    so = jnp.dot(qc, Wso_ref[...], preferred_element_type=jnp.float32) + bso_ref[...]
    awl = jnp.dot(qc, Waw_ref[...], preferred_element_type=jnp.float32) + baw_ref[...]
    m = jnp.maximum(jnp.maximum(awl[:, 0:16], awl[:, 16:32]),
                    jnp.maximum(awl[:, 32:48], awl[:, 48:64]))
    e = jnp.exp(awl - jnp.concatenate([m, m, m, m], axis=-1))
    s = e[:, 0:16] + e[:, 16:32] + e[:, 32:48] + e[:, 48:64]
    aw = e / jnp.concatenate([s, s, s, s], axis=-1)

    xf = so[:, 0:64] + xb_ref[...]
    yf = so[:, 64:128] + yb_ref[...]
    x0 = jnp.floor(xf)
    fx = xf - x0
    cb = jnp.clip(x0, 0.0, 98.0)
    wc0 = jnp.where(x0 == cb, 1.0 - fx, jnp.where(x0 + 1.0 == cb, fx, 0.0))
    wc1 = jnp.where(x0 == cb, fx, jnp.where(x0 == cb + 1.0, 1.0 - fx, 0.0))
    y0 = jnp.floor(yf)
    fy = yf - y0
    rb0 = jnp.clip(y0, 0.0, 99.0)
    rb1 = jnp.clip(y0 + 1.0, 0.0, 99.0)
    wr0 = jnp.where(y0 == rb0, 1.0 - fy, 0.0)
    wr1 = jnp.where(y0 + 1.0 == rb1, fy, 0.0)
    idxA = rb0 * 100.0 + cb
    idxB = rb1 * 100.0 + cb
    idx256 = (jnp.dot(idxA, S01_ref[...], preferred_element_type=jnp.float32,
                      precision=lax.Precision.HIGHEST)
              + jnp.dot(idxB, S23_ref[...], preferred_element_type=jnp.float32,
                        precision=lax.Precision.HIGHEST)) * 8.0 + bi_ref[...]
    idx_ref[...] = (idx256 + 0.5).astype(jnp.int32)
    half = 0.5 * aw
    w_ref[...] = (
        jnp.dot(half * wr0 * wc0, SC0_ref[...], preferred_element_type=jnp.float32)
        + jnp.dot(half * wr0 * wc1, SC1_ref[...], preferred_element_type=jnp.float32)
        + jnp.dot(half * wr1 * wc0, SC2_ref[...], preferred_element_type=jnp.float32)
        + jnp.dot(half * wr1 * wc1, SC3_ref[...], preferred_element_type=jnp.float32))


def _tc1_call(q, qp, xb, yb, Wv, bv, Wso, bso, Waw, baw, interpret=False):
    consts = [jnp.asarray(a) for a in
              (_S01, _S23, _SC0, _SC1, _SC2, _SC3, _BIAS_IDX)]
    cspecs = [pl.BlockSpec((64, 256), lambda i: (0, 0))] * 6 + [
        pl.BlockSpec((1, 256), lambda i: (0, 0))]
    return pl.pallas_call(
        _tc1_body,
        grid=(NQB,),
        in_specs=[
            pl.BlockSpec((QB, D), lambda i: (i, 0)),
            pl.BlockSpec((QB, D), lambda i: (i, 0)),
            pl.BlockSpec((QB, 64), lambda i: (i, 0)),
            pl.BlockSpec((QB, 64), lambda i: (i, 0)),
            pl.BlockSpec((D, D), lambda i: (0, 0)),
            pl.BlockSpec((1, D), lambda i: (0, 0)),
            pl.BlockSpec((2 * D, 128), lambda i: (0, 0)),
            pl.BlockSpec((1, 128), lambda i: (0, 0)),
            pl.BlockSpec((2 * D, 64), lambda i: (0, 0)),
            pl.BlockSpec((1, 64), lambda i: (0, 0)),
        ] + cspecs,
        out_specs=[
            pl.BlockSpec((QB, D), lambda i: (i, 0)),
            pl.BlockSpec((QB, D), lambda i: (i, 0)),
            pl.BlockSpec((QB, D), lambda i: (i, 0)),
        ],
        out_shape=[
            jax.ShapeDtypeStruct((NQ, D), jnp.float32),
            jax.ShapeDtypeStruct((NQ, D), jnp.int32),
            jax.ShapeDtypeStruct((NQ, D), jnp.float32),
        ],
        interpret=interpret,
    )(q, qp, xb, yb, Wv, bv, Wso, bso, Waw, baw, *consts)


def _splat(vec, lane):
    return lax.gather(
        vec, jnp.full((16, 1), lane, jnp.int32),
        dimension_numbers=lax.GatherDimensionNumbers(
            offset_dims=(), collapsed_slice_dims=(0,), start_index_map=(0,)),
        slice_sizes=(1,),
        mode=lax.GatherScatterMode.PROMISE_IN_BOUNDS)


def _sc_body(vt_hbm, idx_hbm, w_hbm, out_hbm,
             idx0, idx1, idx2, w0, w1, w2, r0, r1, r2, out_v,
             sio0, sio1, sio2, sg0, sg1, sg2):
    idxs = (idx0, idx1, idx2)
    ws = (w0, w1, w2)
    rows = (r0, r1, r2)
    sio = (sio0, sio1, sio2)
    sg = (sg0, sg1, sg2)
    wid = lax.axis_index("s") * 2 + lax.axis_index("c")
    c0 = (625 * wid) // 8

    def chunk_q(c):
        return jnp.minimum(c0 + c, NCHUNK - 1) * CQ

    def start_io(c, b):
        qs = chunk_q(c)
        pltpu.async_copy(idx_hbm.at[pl.ds(qs, CQ), :], idxs[b], sio[b])
        pltpu.async_copy(w_hbm.at[pl.ds(qs, CQ), :], ws[b], sio[b])

    def wait_io(b):
        pltpu.make_async_copy(idx_hbm.at[pl.ds(0, CQ), :], idxs[b], sio[b]).wait()
        pltpu.make_async_copy(w_hbm.at[pl.ds(0, CQ), :], ws[b], sio[b]).wait()

    def fire_g(b):
        for qi in range(CQ):
            for hf in range(2):
                pltpu.async_copy(
                    vt_hbm.at[idxs[b].at[qi, pl.ds(hf * 128, 128)]],
                    rows[b].at[pl.ds((qi * 2 + hf) * 128, 128)], sg[b])

    def wait_g(b):
        for qi in range(CQ):
            for hf in range(2):
                pltpu.make_async_copy(
                    vt_hbm.at[idxs[b].at[qi, pl.ds(hf * 128, 128)]],
                    rows[b].at[pl.ds((qi * 2 + hf) * 128, 128)], sg[b]).wait()

    def compute(c, b):
        rr = rows[b]
        wref = ws[b]

        def q_loop(qi, c2):
            base = qi * D
            for h in range(H):
                wv0 = wref[qi, pl.ds(h * HD, 16)]
                wv1 = wref[qi, pl.ds(h * HD + 16, 16)]
                acc0 = jnp.zeros((16,), jnp.float32)
                acc1 = jnp.zeros((16,), jnp.float32)
                for k in range(RPQ):
                    wsp = _splat(wv0 if k < 16 else wv1, k % 16)
                    acc0 = acc0 + wsp * rr[base + h * HD + k, pl.ds(0, 16)]
                    acc1 = acc1 + wsp * rr[base + h * HD + k, pl.ds(16, 16)]
                out_v[qi, pl.ds(h * HD, 16)] = acc0
                out_v[qi, pl.ds(h * HD + 16, 16)] = acc1
            return c2

        lax.fori_loop(0, CQ, q_loop, 0)
        pltpu.sync_copy(out_v, out_hbm.at[pl.ds(chunk_q(c), CQ), :])

    start_io(0, 0)
    wait_io(0)
    fire_g(0)
    start_io(1, 1)

    def body3(i, carry):
        c = 3 * i
        wait_io(1); fire_g(1); start_io(c + 2, 2)
        wait_g(0); compute(c, 0)
        wait_io(2); fire_g(2); start_io(c + 3, 0)
        wait_g(1); compute(c + 1, 1)
        wait_io(0); fire_g(0); start_io(c + 4, 1)
        wait_g(2); compute(c + 2, 2)
        return carry

    lax.fori_loop(0, (NCHT - 2) // 3, body3, 0)
    wait_io(1)
    fire_g(1)
    wait_g(0)
    compute(NCHT - 2, 0)
    wait_g(1)
    compute(NCHT - 1, 1)


@functools.lru_cache(maxsize=1)
def _sc_call():
    return pl.kernel(
        _sc_body,
        out_type=jax.ShapeDtypeStruct((NQ, D), jnp.float32),
        mesh=plsc.VectorSubcoreMesh(core_axis_name="c", subcore_axis_name="s"),
        scratch_types=(
            [pltpu.VMEM((CQ, D), jnp.int32)] * 3
            + [pltpu.VMEM((CQ, D), jnp.float32)] * 3
            + [pltpu.VMEM((ROWSC, HD), jnp.float32)] * 3
            + [pltpu.VMEM((CQ, D), jnp.float32)]
            + [pltpu.SemaphoreType.DMA] * 6
        ),
        compiler_params=pltpu.CompilerParams(use_tc_tiling_on_sc=False),
    )


def _tc2_body(q_ref, s_ref, Wo_ref, bo_ref, o_ref):
    o_ref[...] = (q_ref[...] + bo_ref[...]
                  + jnp.dot(s_ref[...], Wo_ref[...],
                            preferred_element_type=jnp.float32))


def _tc2_call(q, sout, Wo, bo, interpret=False):
    return pl.pallas_call(
        _tc2_body,
        grid=(NQB,),
        in_specs=[
            pl.BlockSpec((QB, D), lambda i: (i, 0)),
            pl.BlockSpec((QB, D), lambda i: (i, 0)),
            pl.BlockSpec((D, D), lambda i: (0, 0)),
            pl.BlockSpec((1, D), lambda i: (0, 0)),
        ],
        out_specs=pl.BlockSpec((QB, D), lambda i: (i, 0)),
        out_shape=jax.ShapeDtypeStruct((NQ, D), jnp.float32),
        interpret=interpret,
    )(q, sout, Wo, bo)


def kernel(query, query_pos, reference_points, spatial_shapes,
           level_start_index, W_value, b_value, W_so, b_so, W_aw, b_aw,
           W_out, b_out):
    q2 = query[0]
    qp2 = query_pos[0]
    refx = reference_points[:, :, 0, 0].T
    refy = reference_points[:, :, 0, 1].T
    xb = refx[:, _BSEL] * 100.0 - 0.5
    yb = refy[:, _BSEL] * 100.0 - 0.5
    Wso_p = W_so[:, _PERM_SO]
    bso_p = b_so[_PERM_SO][None, :]
    Waw_p = W_aw[:, _PERM_AW]
    baw_p = b_aw[_PERM_AW][None, :]
    v, idx, w = _tc1_call(q2, qp2, xb, yb, W_value, b_value[None, :],
                          Wso_p, bso_p, Waw_p, baw_p)
    vtab = v.reshape(NQ * H, HD)
    sout = _sc_call()(vtab, idx, w)
    out = _tc2_call(q2, sout, W_out, b_out[None, :])
    return out[None]
